# baseline (speedup 1.0000x reference)
import jax
import jax.numpy as jnp
from jax import lax
from jax.experimental import pallas as pl
from jax.experimental.pallas import tpu as pltpu
from jax.experimental.pallas import tpu_sc as plsc

OUT = 7
C = 256
CW = C // 2
M = 1024
NC, NS, L = 2, 16, 16
NW = NC * NS
BOX_PER_W = M // NW
NPOS = OUT * OUT
SLOTS = 52
TAB = 2 * SLOTS
WTAB = 4 * 56
ETAB = 128
OUT_WORDS = C * NPOS
NPIX = 43520
NPAIR = NPIX // 2

_GRID = tuple((i + 0.5) / OUT for i in range(OUT))


def _sc_body(table, boxes_t, out_flat, coords, idx_all, w_all, e_all,
             rows_a, rows_b, out_v, sem_a, sem_b):
    wid = lax.axis_index("s") * NC + lax.axis_index("c")
    box0 = wid * BOX_PER_W
    iota = lax.iota(jnp.int32, L)
    zeros_i = jnp.zeros((L,), jnp.int32)
    ones_i = jnp.full((L,), 1, jnp.int32)

    @pl.loop(0, 2)
    def _groups(g):
        gb = box0 + g * L
        for c4 in range(4):
            pltpu.sync_copy(boxes_t.at[c4, pl.ds(gb, L)], coords.at[c4])
        x1 = coords[0]
        y1 = coords[1]
        x2 = coords[2]
        y2 = coords[3]
        area = (x2 - x1) * (y2 - y1)
        lvm2 = (jnp.where(area >= 12544.0, ones_i, zeros_i)
                + jnp.where(area >= 50176.0, ones_i, zeros_i)
                + jnp.where(area >= 200704.0, ones_i, zeros_i))
        stridef = jnp.left_shift(jnp.full((L,), 4, jnp.int32),
                                 lvm2).astype(jnp.float32)
        wi = jnp.right_shift(jnp.full((L,), 128, jnp.int32), lvm2)
        hw = wi * wi
        base_rows = jnp.where(
            lvm2 == 0, zeros_i,
            jnp.where(lvm2 == 1, jnp.full((L,), 32768, jnp.int32),
                      jnp.where(lvm2 == 2, jnp.full((L,), 40960, jnp.int32),
                                jnp.full((L,), 43008, jnp.int32))))
        bvec = jnp.full((L,), gb, jnp.int32)
        rowbase = base_rows + jnp.where(bvec >= 512, hw, zeros_i)
        wim1 = wi - ones_i

        x1s = x1 / stridef
        x2s = x2 / stridef
        y1s = y1 / stridef
        y2s = y2 / stridef
        starts, e0l, e1l, wxl, omwxl = [], [], [], [], []
        rb0, rb1, wyl, omwyl = [], [], [], []
        for o in range(OUT):
            t = _GRID[o]
            px = x1s + t * (x2s - x1s)
            x0t = px.astype(jnp.int32)
            wx = px - x0t.astype(jnp.float32)
            start = jnp.minimum(x0t, wim1 - ones_i)
            e0l.append((jnp.minimum(x0t, wim1) - start) * CW)
            e1l.append((jnp.minimum(x0t + 1, wim1) - start) * CW)
            starts.append(start)
            wxl.append(wx)
            omwxl.append(1.0 - wx)
            py = y1s + t * (y2s - y1s)
            y0t = py.astype(jnp.int32)
            wy = py - y0t.astype(jnp.float32)
            rb0.append(rowbase + jnp.minimum(y0t, wim1) * wi)
            rb1.append(rowbase + jnp.minimum(y0t + 1, wim1) * wi)
            wyl.append(wy)
            omwyl.append(1.0 - wy)

        tb_i = (g * L + iota) * TAB
        tb_w = (g * L + iota) * WTAB
        tb_e = (g * L + iota) * ETAB
        npair_v = jnp.full((L,), NPAIR, jnp.int32)
        p = 0
        for oy in range(OUT):
            for ox in range(OUT):
                g0 = rb0[oy] + starts[ox]
                g1 = rb1[oy] + starts[ox]
                t0 = jnp.right_shift(g0, 1) + (g0 & 1) * npair_v
                t1 = jnp.right_shift(g1, 1) + (g1 & 1) * npair_v
                plsc.store_scatter(idx_all, [tb_i + p], t0)
                plsc.store_scatter(idx_all, [tb_i + (SLOTS + p)], t1)
                wvals = (omwyl[oy] * omwxl[ox], omwyl[oy] * wxl[ox],
                         wyl[oy] * omwxl[ox], wyl[oy] * wxl[ox])
                for c4, wv in enumerate(wvals):
                    plsc.store_scatter(w_all, [tb_w + (c4 * 56 + p)], wv)
                if oy == 0:
                    plsc.store_scatter(e_all, [tb_e + ox], e0l[ox])
                    plsc.store_scatter(e_all, [tb_e + (64 + ox)], e1l[ox])
                p += 1
        for p in range(NPOS, SLOTS):
            plsc.store_scatter(idx_all, [tb_i + p], zeros_i)
            plsc.store_scatter(idx_all, [tb_i + (SLOTS + p)], zeros_i)

    ce = [(jnp.full((L,), k * 32, jnp.int32) + 2 * iota) * NPOS
          for k in range(C // 32)]
    co = [(jnp.full((L,), k * 32 + 1, jnp.int32) + 2 * iota) * NPOS
          for k in range(C // 32)]

    def gather(b, rows_ref, sem):
        pltpu.async_copy(table.at[idx_all.at[pl.ds(b * TAB, TAB)]],
                         rows_ref, sem)

    def drain(rows_ref, sem):
        pltpu.make_async_copy(table.at[pl.ds(0, TAB)], rows_ref, sem).wait()

    def compute(b, rows_ref):
        wbase = jnp.full((L,), b * WTAB, jnp.int32)
        ebase = jnp.full((L,), b * ETAB, jnp.int32)

        @pl.loop(0, NPOS)
        def _pos(p):
            wp = wbase + p
            w00 = plsc.load_gather(w_all, [wp])
            w01 = plsc.load_gather(w_all, [wp + 56])
            w10 = plsc.load_gather(w_all, [wp + 2 * 56])
            w11 = plsc.load_gather(w_all, [wp + 3 * 56])
            ox = lax.rem(p, OUT)
            c0 = plsc.load_gather(e_all, [ebase + ox]) + iota
            c1 = plsc.load_gather(e_all, [ebase + (64 + ox)]) + iota
            r0 = jnp.full((L,), p, jnp.int32)
            r1 = jnp.full((L,), SLOTS + p, jnp.int32)
            for k in range(C // 32):
                k16 = k * L
                v00 = plsc.load_gather(rows_ref, [r0, c0 + k16])
                v01 = plsc.load_gather(rows_ref, [r0, c1 + k16])
                v10 = plsc.load_gather(rows_ref, [r1, c0 + k16])
                v11 = plsc.load_gather(rows_ref, [r1, c1 + k16])
                e0, o0 = plsc.unpack(
                    plsc.bitcast(v00, jnp.bfloat16),
                    format=plsc.PackFormat.INTERLEAVED,
                    preferred_element_type=jnp.float32)
                e1, o1 = plsc.unpack(
                    plsc.bitcast(v01, jnp.bfloat16),
                    format=plsc.PackFormat.INTERLEAVED,
                    preferred_element_type=jnp.float32)
                e2, o2 = plsc.unpack(
                    plsc.bitcast(v10, jnp.bfloat16),
                    format=plsc.PackFormat.INTERLEAVED,
                    preferred_element_type=jnp.float32)
                e3, o3 = plsc.unpack(
                    plsc.bitcast(v11, jnp.bfloat16),
                    format=plsc.PackFormat.INTERLEAVED,
                    preferred_element_type=jnp.float32)
                acc_e = e0 * w00 + e1 * w01 + e2 * w10 + e3 * w11
                acc_o = o0 * w00 + o1 * w01 + o2 * w10 + o3 * w11
                plsc.store_scatter(out_v, [ce[k] + p], acc_e)
                plsc.store_scatter(out_v, [co[k] + p], acc_o)

        pltpu.sync_copy(
            out_v, out_flat.at[pl.ds((box0 + b) * OUT_WORDS, OUT_WORDS)])

    gather(0, rows_a, sem_a)

    @pl.loop(0, BOX_PER_W, step=2)
    def _pairs(b0):
        b1 = b0 + 1
        gather(b1, rows_b, sem_b)
        drain(rows_a, sem_a)
        compute(b0, rows_a)
        bn = jnp.minimum(b0 + 2, BOX_PER_W - 1)
        gather(bn, rows_a, sem_a)
        drain(rows_b, sem_b)
        compute(b1, rows_b)

    drain(rows_a, sem_a)


_mesh = plsc.VectorSubcoreMesh(
    core_axis_name="c", subcore_axis_name="s", num_cores=NC, num_subcores=NS)

_run = pl.kernel(
    _sc_body,
    out_type=jax.ShapeDtypeStruct((M * OUT_WORDS,), jnp.float32),
    mesh=_mesh,
    compiler_params=pltpu.CompilerParams(needs_layout_passes=False),
    scratch_types=[
        pltpu.VMEM((4, L), jnp.float32),
        pltpu.VMEM((BOX_PER_W * TAB,), jnp.int32),
        pltpu.VMEM((BOX_PER_W * WTAB,), jnp.float32),
        pltpu.VMEM((BOX_PER_W * ETAB,), jnp.int32),
        pltpu.VMEM((TAB, 2 * CW), jnp.int32),
        pltpu.VMEM((TAB, 2 * CW), jnp.int32),
        pltpu.VMEM((OUT_WORDS,), jnp.float32),
        pltpu.SemaphoreType.DMA,
        pltpu.SemaphoreType.DMA,
    ],
)


@jax.jit
def kernel(fm2, fm3, fm4, fm5, boxes1, boxes2):
    tabs = [jnp.transpose(fm.astype(jnp.bfloat16), (0, 2, 3, 1)).reshape(-1, C)
            for fm in (fm2, fm3, fm4, fm5)]
    flat = jax.lax.bitcast_convert_type(
        jnp.concatenate(tabs, axis=0).reshape(NPIX, CW, 2),
        jnp.int32)
    words = flat.reshape(-1)
    even = words.reshape(NPAIR, 2 * CW)
    odd = jnp.concatenate(
        [words[CW:], jnp.zeros((CW,), jnp.int32)]).reshape(NPAIR, 2 * CW)
    table = jnp.concatenate([even, odd], axis=0)
    boxes_t = jnp.concatenate([boxes1, boxes2], axis=0).T
    out_flat = _run(table, boxes_t)
    return out_flat.reshape(M, C, OUT, OUT)


# speedup vs baseline: 1.0012x; 1.0012x over previous
import jax
import jax.numpy as jnp
from jax import lax
from jax.experimental import pallas as pl
from jax.experimental.pallas import tpu as pltpu
from jax.experimental.pallas import tpu_sc as plsc

OUT = 7
C = 256
CW = C // 2
M = 1024
NC, NS, L = 2, 16, 16
NW = NC * NS
BOX_PER_W = M // NW
NPOS = OUT * OUT
SLOTS = 52
TAB = 2 * SLOTS
WTAB = 4 * 56
ETAB = 128
OUT_WORDS = C * NPOS
NPIX = 43520
NPAIR = NPIX // 2

_GRID = tuple((i + 0.5) / OUT for i in range(OUT))


def _sc_body(table, b1f, b2f, out_flat, coords_f, idx_all, w_all, e_all,
             rows_a, rows_b, out_v, sem_a, sem_b):
    wid = lax.axis_index("s") * NC + lax.axis_index("c")
    box0 = wid * BOX_PER_W
    iota = lax.iota(jnp.int32, L)
    zeros_i = jnp.zeros((L,), jnp.int32)
    ones_i = jnp.full((L,), 1, jnp.int32)

    @pl.loop(0, 2)
    def _groups(g):
        gb = box0 + g * L

        @pl.when(gb < 512)
        def _():
            pltpu.sync_copy(b1f.at[pl.ds(gb * 4, 4 * L)], coords_f)

        @pl.when(gb >= 512)
        def _():
            pltpu.sync_copy(b2f.at[pl.ds((gb - 512) * 4, 4 * L)], coords_f)

        ci = iota * 4
        x1 = plsc.load_gather(coords_f, [ci])
        y1 = plsc.load_gather(coords_f, [ci + 1])
        x2 = plsc.load_gather(coords_f, [ci + 2])
        y2 = plsc.load_gather(coords_f, [ci + 3])
        area = (x2 - x1) * (y2 - y1)
        lvm2 = (jnp.where(area >= 12544.0, ones_i, zeros_i)
                + jnp.where(area >= 50176.0, ones_i, zeros_i)
                + jnp.where(area >= 200704.0, ones_i, zeros_i))
        stridef = jnp.left_shift(jnp.full((L,), 4, jnp.int32),
                                 lvm2).astype(jnp.float32)
        wi = jnp.right_shift(jnp.full((L,), 128, jnp.int32), lvm2)
        hw = wi * wi
        base_rows = jnp.where(
            lvm2 == 0, zeros_i,
            jnp.where(lvm2 == 1, jnp.full((L,), 32768, jnp.int32),
                      jnp.where(lvm2 == 2, jnp.full((L,), 40960, jnp.int32),
                                jnp.full((L,), 43008, jnp.int32))))
        bvec = jnp.full((L,), gb, jnp.int32)
        rowbase = base_rows + jnp.where(bvec >= 512, hw, zeros_i)
        wim1 = wi - ones_i

        x1s = x1 / stridef
        x2s = x2 / stridef
        y1s = y1 / stridef
        y2s = y2 / stridef
        starts, e0l, e1l, wxl, omwxl = [], [], [], [], []
        rb0, rb1, wyl, omwyl = [], [], [], []
        for o in range(OUT):
            t = _GRID[o]
            px = x1s + t * (x2s - x1s)
            x0t = px.astype(jnp.int32)
            wx = px - x0t.astype(jnp.float32)
            start = jnp.minimum(x0t, wim1 - ones_i)
            e0l.append((jnp.minimum(x0t, wim1) - start) * CW)
            e1l.append((jnp.minimum(x0t + 1, wim1) - start) * CW)
            starts.append(start)
            wxl.append(wx)
            omwxl.append(1.0 - wx)
            py = y1s + t * (y2s - y1s)
            y0t = py.astype(jnp.int32)
            wy = py - y0t.astype(jnp.float32)
            rb0.append(rowbase + jnp.minimum(y0t, wim1) * wi)
            rb1.append(rowbase + jnp.minimum(y0t + 1, wim1) * wi)
            wyl.append(wy)
            omwyl.append(1.0 - wy)

        tb_i = (g * L + iota) * TAB
        tb_w = (g * L + iota) * WTAB
        tb_e = (g * L + iota) * ETAB
        npair_v = jnp.full((L,), NPAIR, jnp.int32)
        p = 0
        for oy in range(OUT):
            for ox in range(OUT):
                g0 = rb0[oy] + starts[ox]
                g1 = rb1[oy] + starts[ox]
                t0 = jnp.right_shift(g0, 1) + (g0 & 1) * npair_v
                t1 = jnp.right_shift(g1, 1) + (g1 & 1) * npair_v
                plsc.store_scatter(idx_all, [tb_i + p], t0)
                plsc.store_scatter(idx_all, [tb_i + (SLOTS + p)], t1)
                wvals = (omwyl[oy] * omwxl[ox], omwyl[oy] * wxl[ox],
                         wyl[oy] * omwxl[ox], wyl[oy] * wxl[ox])
                for c4, wv in enumerate(wvals):
                    plsc.store_scatter(w_all, [tb_w + (c4 * 56 + p)], wv)
                if oy == 0:
                    plsc.store_scatter(e_all, [tb_e + ox], e0l[ox])
                    plsc.store_scatter(e_all, [tb_e + (64 + ox)], e1l[ox])
                p += 1
        for p in range(NPOS, SLOTS):
            plsc.store_scatter(idx_all, [tb_i + p], zeros_i)
            plsc.store_scatter(idx_all, [tb_i + (SLOTS + p)], zeros_i)

    ce = [(jnp.full((L,), k * 32, jnp.int32) + 2 * iota) * NPOS
          for k in range(C // 32)]
    co = [(jnp.full((L,), k * 32 + 1, jnp.int32) + 2 * iota) * NPOS
          for k in range(C // 32)]

    def gather(b, rows_ref, sem):
        pltpu.async_copy(table.at[idx_all.at[pl.ds(b * TAB, TAB)]],
                         rows_ref, sem)

    def drain(rows_ref, sem):
        pltpu.make_async_copy(table.at[pl.ds(0, TAB)], rows_ref, sem).wait()

    def compute(b, rows_ref):
        wbase = jnp.full((L,), b * WTAB, jnp.int32)
        ebase = jnp.full((L,), b * ETAB, jnp.int32)

        @pl.loop(0, NPOS)
        def _pos(p):
            wp = wbase + p
            w00 = plsc.load_gather(w_all, [wp])
            w01 = plsc.load_gather(w_all, [wp + 56])
            w10 = plsc.load_gather(w_all, [wp + 2 * 56])
            w11 = plsc.load_gather(w_all, [wp + 3 * 56])
            ox = lax.rem(p, OUT)
            c0 = plsc.load_gather(e_all, [ebase + ox]) + iota
            c1 = plsc.load_gather(e_all, [ebase + (64 + ox)]) + iota
            r0 = jnp.full((L,), p, jnp.int32)
            r1 = jnp.full((L,), SLOTS + p, jnp.int32)
            for k in range(C // 32):
                k16 = k * L
                v00 = plsc.load_gather(rows_ref, [r0, c0 + k16])
                v01 = plsc.load_gather(rows_ref, [r0, c1 + k16])
                v10 = plsc.load_gather(rows_ref, [r1, c0 + k16])
                v11 = plsc.load_gather(rows_ref, [r1, c1 + k16])
                e0, o0 = plsc.unpack(
                    plsc.bitcast(v00, jnp.bfloat16),
                    format=plsc.PackFormat.INTERLEAVED,
                    preferred_element_type=jnp.float32)
                e1, o1 = plsc.unpack(
                    plsc.bitcast(v01, jnp.bfloat16),
                    format=plsc.PackFormat.INTERLEAVED,
                    preferred_element_type=jnp.float32)
                e2, o2 = plsc.unpack(
                    plsc.bitcast(v10, jnp.bfloat16),
                    format=plsc.PackFormat.INTERLEAVED,
                    preferred_element_type=jnp.float32)
                e3, o3 = plsc.unpack(
                    plsc.bitcast(v11, jnp.bfloat16),
                    format=plsc.PackFormat.INTERLEAVED,
                    preferred_element_type=jnp.float32)
                acc_e = e0 * w00 + e1 * w01 + e2 * w10 + e3 * w11
                acc_o = o0 * w00 + o1 * w01 + o2 * w10 + o3 * w11
                plsc.store_scatter(out_v, [ce[k] + p], acc_e)
                plsc.store_scatter(out_v, [co[k] + p], acc_o)

        pltpu.sync_copy(
            out_v, out_flat.at[pl.ds((box0 + b) * OUT_WORDS, OUT_WORDS)])

    gather(0, rows_a, sem_a)

    @pl.loop(0, BOX_PER_W, step=2)
    def _pairs(b0):
        b1 = b0 + 1
        gather(b1, rows_b, sem_b)
        drain(rows_a, sem_a)
        compute(b0, rows_a)
        bn = jnp.minimum(b0 + 2, BOX_PER_W - 1)
        gather(bn, rows_a, sem_a)
        drain(rows_b, sem_b)
        compute(b1, rows_b)

    drain(rows_a, sem_a)


_mesh = plsc.VectorSubcoreMesh(
    core_axis_name="c", subcore_axis_name="s", num_cores=NC, num_subcores=NS)

_run = pl.kernel(
    _sc_body,
    out_type=jax.ShapeDtypeStruct((M * OUT_WORDS,), jnp.float32),
    mesh=_mesh,
    compiler_params=pltpu.CompilerParams(needs_layout_passes=False),
    scratch_types=[
        pltpu.VMEM((4 * L,), jnp.float32),
        pltpu.VMEM((BOX_PER_W * TAB,), jnp.int32),
        pltpu.VMEM((BOX_PER_W * WTAB,), jnp.float32),
        pltpu.VMEM((BOX_PER_W * ETAB,), jnp.int32),
        pltpu.VMEM((TAB, 2 * CW), jnp.int32),
        pltpu.VMEM((TAB, 2 * CW), jnp.int32),
        pltpu.VMEM((OUT_WORDS,), jnp.float32),
        pltpu.SemaphoreType.DMA,
        pltpu.SemaphoreType.DMA,
    ],
)


@jax.jit
def kernel(fm2, fm3, fm4, fm5, boxes1, boxes2):
    tabs = [jnp.transpose(fm.astype(jnp.bfloat16), (0, 2, 3, 1)).reshape(-1, C)
            for fm in (fm2, fm3, fm4, fm5)]
    flat = jax.lax.bitcast_convert_type(
        jnp.concatenate(tabs, axis=0).reshape(NPIX, CW, 2),
        jnp.int32)
    words = flat.reshape(-1)
    even = words.reshape(NPAIR, 2 * CW)
    odd = jnp.concatenate(
        [words[CW:], jnp.zeros((CW,), jnp.int32)]).reshape(NPAIR, 2 * CW)
    table = jnp.concatenate([even, odd], axis=0)
    out_flat = _run(table, boxes1.reshape(-1), boxes2.reshape(-1))
    return out_flat.reshape(M, C, OUT, OUT)


# speedup vs baseline: 1.0993x; 1.0980x over previous
import jax
import jax.numpy as jnp
from jax import lax
from jax.experimental import pallas as pl
from jax.experimental.pallas import tpu as pltpu
from jax.experimental.pallas import tpu_sc as plsc

OUT = 7
C = 256
CW = C // 2
M = 1024
NC, NS, L = 2, 16, 16
NW = NC * NS
BOX_PER_W = M // NW
NPOS = OUT * OUT
SLOTS = 52
TAB = 2 * SLOTS
WTAB = 4 * 56
ETAB = 128
OUT_WORDS = C * NPOS
NPIX = 43520
NPAIR = NPIX // 2

_GRID = tuple((i + 0.5) / OUT for i in range(OUT))


def _sc_body(table, b1f, b2f, out_flat, coords_f, idx_all, w_all, e_all,
             rows_a, rows_b, out_v, sem_a, sem_b):
    wid = lax.axis_index("s") * NC + lax.axis_index("c")
    box0 = wid * BOX_PER_W
    iota = lax.iota(jnp.int32, L)
    zeros_i = jnp.zeros((L,), jnp.int32)
    ones_i = jnp.full((L,), 1, jnp.int32)

    @pl.loop(0, 2)
    def _groups(g):
        gb = box0 + g * L

        @pl.when(gb < 512)
        def _():
            pltpu.sync_copy(b1f.at[pl.ds(gb * 4, 4 * L)], coords_f)

        @pl.when(gb >= 512)
        def _():
            pltpu.sync_copy(b2f.at[pl.ds((gb - 512) * 4, 4 * L)], coords_f)

        ci = iota * 4
        x1 = plsc.load_gather(coords_f, [ci])
        y1 = plsc.load_gather(coords_f, [ci + 1])
        x2 = plsc.load_gather(coords_f, [ci + 2])
        y2 = plsc.load_gather(coords_f, [ci + 3])
        area = (x2 - x1) * (y2 - y1)
        lvm2 = (jnp.where(area >= 12544.0, ones_i, zeros_i)
                + jnp.where(area >= 50176.0, ones_i, zeros_i)
                + jnp.where(area >= 200704.0, ones_i, zeros_i))
        stridef = jnp.left_shift(jnp.full((L,), 4, jnp.int32),
                                 lvm2).astype(jnp.float32)
        wi = jnp.right_shift(jnp.full((L,), 128, jnp.int32), lvm2)
        hw = wi * wi
        base_rows = jnp.where(
            lvm2 == 0, zeros_i,
            jnp.where(lvm2 == 1, jnp.full((L,), 32768, jnp.int32),
                      jnp.where(lvm2 == 2, jnp.full((L,), 40960, jnp.int32),
                                jnp.full((L,), 43008, jnp.int32))))
        bvec = jnp.full((L,), gb, jnp.int32)
        rowbase = base_rows + jnp.where(bvec >= 512, hw, zeros_i)
        wim1 = wi - ones_i

        x1s = x1 / stridef
        x2s = x2 / stridef
        y1s = y1 / stridef
        y2s = y2 / stridef
        starts, e0l, e1l, wxl, omwxl = [], [], [], [], []
        rb0, rb1, wyl, omwyl = [], [], [], []
        for o in range(OUT):
            t = _GRID[o]
            px = x1s + t * (x2s - x1s)
            x0t = px.astype(jnp.int32)
            wx = px - x0t.astype(jnp.float32)
            start = jnp.minimum(x0t, wim1 - ones_i)
            e0l.append((jnp.minimum(x0t, wim1) - start) * CW)
            e1l.append((jnp.minimum(x0t + 1, wim1) - start) * CW)
            starts.append(start)
            wxl.append(wx)
            omwxl.append(1.0 - wx)
            py = y1s + t * (y2s - y1s)
            y0t = py.astype(jnp.int32)
            wy = py - y0t.astype(jnp.float32)
            rb0.append(rowbase + jnp.minimum(y0t, wim1) * wi)
            rb1.append(rowbase + jnp.minimum(y0t + 1, wim1) * wi)
            wyl.append(wy)
            omwyl.append(1.0 - wy)

        tb_i = (g * L + iota) * TAB
        tb_w = (g * L + iota) * WTAB
        tb_e = (g * L + iota) * ETAB
        npair_v = jnp.full((L,), NPAIR, jnp.int32)
        p = 0
        for oy in range(OUT):
            for ox in range(OUT):
                g0 = rb0[oy] + starts[ox]
                g1 = rb1[oy] + starts[ox]
                t0 = jnp.right_shift(g0, 1) + (g0 & 1) * npair_v
                t1 = jnp.right_shift(g1, 1) + (g1 & 1) * npair_v
                plsc.store_scatter(idx_all, [tb_i + p], t0)
                plsc.store_scatter(idx_all, [tb_i + (SLOTS + p)], t1)
                wvals = (omwyl[oy] * omwxl[ox], omwyl[oy] * wxl[ox],
                         wyl[oy] * omwxl[ox], wyl[oy] * wxl[ox])
                for c4, wv in enumerate(wvals):
                    plsc.store_scatter(w_all, [tb_w + (c4 * 56 + p)], wv)
                if oy == 0:
                    plsc.store_scatter(e_all, [tb_e + ox], e0l[ox])
                    plsc.store_scatter(e_all, [tb_e + (64 + ox)], e1l[ox])
                p += 1
        for p in range(NPOS, SLOTS):
            plsc.store_scatter(idx_all, [tb_i + p], zeros_i)
            plsc.store_scatter(idx_all, [tb_i + (SLOTS + p)], zeros_i)

    ce = [(jnp.full((L,), k * 32, jnp.int32) + 2 * iota) * NPOS
          for k in range(C // 32)]
    co = [(jnp.full((L,), k * 32 + 1, jnp.int32) + 2 * iota) * NPOS
          for k in range(C // 32)]

    def gather(b, rows_ref, sem):
        pltpu.async_copy(table.at[idx_all.at[pl.ds(b * TAB, TAB)]],
                         rows_ref, sem)

    def drain(rows_ref, sem):
        pltpu.make_async_copy(table.at[pl.ds(0, TAB)], rows_ref, sem).wait()

    def compute(b, rows_ref):
        wbase = jnp.full((L,), b * WTAB, jnp.int32)
        ebase = jnp.full((L,), b * ETAB, jnp.int32)

        @pl.loop(0, NPOS)
        def _pos(p):
            wp = wbase + p
            w00 = plsc.load_gather(w_all, [wp])
            w01 = plsc.load_gather(w_all, [wp + 56])
            w10 = plsc.load_gather(w_all, [wp + 2 * 56])
            w11 = plsc.load_gather(w_all, [wp + 3 * 56])
            ox = lax.rem(p, OUT)
            c0 = plsc.load_gather(e_all, [ebase + ox]) + iota
            c1 = plsc.load_gather(e_all, [ebase + (64 + ox)]) + iota
            r0 = jnp.full((L,), p, jnp.int32)
            r1 = jnp.full((L,), SLOTS + p, jnp.int32)
            for k in range(C // 32):
                k16 = k * L
                v00 = plsc.load_gather(rows_ref, [r0, c0 + k16])
                v01 = plsc.load_gather(rows_ref, [r0, c1 + k16])
                v10 = plsc.load_gather(rows_ref, [r1, c0 + k16])
                v11 = plsc.load_gather(rows_ref, [r1, c1 + k16])
                e0, o0 = plsc.unpack(
                    plsc.bitcast(v00, jnp.bfloat16),
                    format=plsc.PackFormat.INTERLEAVED,
                    preferred_element_type=jnp.float32)
                e1, o1 = plsc.unpack(
                    plsc.bitcast(v01, jnp.bfloat16),
                    format=plsc.PackFormat.INTERLEAVED,
                    preferred_element_type=jnp.float32)
                e2, o2 = plsc.unpack(
                    plsc.bitcast(v10, jnp.bfloat16),
                    format=plsc.PackFormat.INTERLEAVED,
                    preferred_element_type=jnp.float32)
                e3, o3 = plsc.unpack(
                    plsc.bitcast(v11, jnp.bfloat16),
                    format=plsc.PackFormat.INTERLEAVED,
                    preferred_element_type=jnp.float32)
                acc_e = e0 * w00 + e1 * w01 + e2 * w10 + e3 * w11
                acc_o = o0 * w00 + o1 * w01 + o2 * w10 + o3 * w11
                plsc.store_scatter(out_v, [ce[k] + p], acc_e)
                plsc.store_scatter(out_v, [co[k] + p], acc_o)

        pltpu.sync_copy(
            out_v, out_flat.at[pl.ds((box0 + b) * OUT_WORDS, OUT_WORDS)])

    gather(0, rows_a, sem_a)

    @pl.loop(0, BOX_PER_W, step=2)
    def _pairs(b0):
        b1 = b0 + 1
        gather(b1, rows_b, sem_b)
        drain(rows_a, sem_a)
        compute(b0, rows_a)
        bn = jnp.minimum(b0 + 2, BOX_PER_W - 1)
        gather(bn, rows_a, sem_a)
        drain(rows_b, sem_b)
        compute(b1, rows_b)

    drain(rows_a, sem_a)


_mesh = plsc.VectorSubcoreMesh(
    core_axis_name="c", subcore_axis_name="s", num_cores=NC, num_subcores=NS)

_run = pl.kernel(
    _sc_body,
    out_type=jax.ShapeDtypeStruct((M * OUT_WORDS,), jnp.float32),
    mesh=_mesh,
    compiler_params=pltpu.CompilerParams(needs_layout_passes=False),
    scratch_types=[
        pltpu.VMEM((4 * L,), jnp.float32),
        pltpu.VMEM((BOX_PER_W * TAB,), jnp.int32),
        pltpu.VMEM((BOX_PER_W * WTAB,), jnp.float32),
        pltpu.VMEM((BOX_PER_W * ETAB,), jnp.int32),
        pltpu.VMEM((TAB, 2 * CW), jnp.int32),
        pltpu.VMEM((TAB, 2 * CW), jnp.int32),
        pltpu.VMEM((OUT_WORDS,), jnp.float32),
        pltpu.SemaphoreType.DMA,
        pltpu.SemaphoreType.DMA,
    ],
)


@jax.jit
def kernel(fm2, fm3, fm4, fm5, boxes1, boxes2):
    evens, odds = [], []
    zpad = jnp.zeros((CW,), jnp.int32)
    for fm in (fm2, fm3, fm4, fm5):
        t = jnp.transpose(fm.astype(jnp.bfloat16), (0, 2, 3, 1))
        w = jax.lax.bitcast_convert_type(
            t.reshape(-1, CW, 2), jnp.int32).reshape(-1)
        evens.append(w.reshape(-1, 2 * CW))
        odds.append(jnp.concatenate([w[CW:], zpad]).reshape(-1, 2 * CW))
    table = jnp.concatenate(evens + odds, axis=0)
    out_flat = _run(table, boxes1.reshape(-1), boxes2.reshape(-1))
    return out_flat.reshape(M, C, OUT, OUT)
